# baseline (speedup 1.0000x reference)
import jax
import jax.numpy as jnp
from jax import lax
from jax.experimental import pallas as pl
from jax.experimental.pallas import tpu as pltpu
from jax.experimental.pallas import tpu_sc as plsc

B = 16384
D = 64
NC = 2
NS = 16
NW = NC * NS
BPW = B // NW
L = 16
CHUNK = 128
NCH = BPW // CHUNK
NG = CHUNK // L


def _sc_body(u_ids_hbm, i_ids_hbm, ent_map_hbm, user_hbm, item_hbm, ent_hbm,
             out_hbm, uid_v, iid_v, eid_v, upid_v, ipid_v, epid_v,
             u_rows, i_rows, e_rows, out_v, sem_u, sem_i, sem_e):
    wid = lax.axis_index("s") * NC + lax.axis_index("c")
    base = wid * BPW

    pltpu.sync_copy(u_ids_hbm.at[pl.ds(base, BPW)], uid_v)
    pltpu.sync_copy(i_ids_hbm.at[pl.ds(base, BPW)], iid_v)
    pltpu.async_copy(ent_map_hbm.at[iid_v], eid_v, sem_e).wait()

    def pid_body(g, _):
        sl = pl.ds(g * L, L)
        upid_v[sl] = lax.shift_right_logical(uid_v[sl], 1)
        ipid_v[sl] = lax.shift_right_logical(iid_v[sl], 1)
        epid_v[sl] = lax.shift_right_logical(eid_v[sl], 1)
        return 0

    lax.fori_loop(0, BPW // L, pid_body, 0)

    lane = lax.iota(jnp.int32, L)

    def chunk_body(ch, _):
        c0 = ch * CHUNK
        cp_u = pltpu.async_copy(
            user_hbm.at[upid_v.at[pl.ds(c0, CHUNK)]], u_rows, sem_u)
        cp_i = pltpu.async_copy(
            item_hbm.at[ipid_v.at[pl.ds(c0, CHUNK)]], i_rows, sem_i)
        cp_e = pltpu.async_copy(
            ent_hbm.at[epid_v.at[pl.ds(c0, CHUNK)]], e_rows, sem_e)
        cp_u.wait()
        cp_i.wait()
        cp_e.wait()

        def group_body(g, _):
            b0 = g * L
            gsl = pl.ds(c0 + b0, L)
            uoff = (uid_v[gsl] & 1) * D
            ioff = (iid_v[gsl] & 1) * D
            eoff = (eid_v[gsl] & 1) * D
            acc = jnp.zeros((L,), jnp.float32)
            for k in range(L):
                b = b0 + k
                uo = uoff[k]
                io = ioff[k]
                eo = eoff[k]
                p = jnp.zeros((L,), jnp.float32)
                for j in range(D // L):
                    u = u_rows[b, pl.ds(uo + j * L, L)]
                    iv = i_rows[b, pl.ds(io + j * L, L)]
                    ev = e_rows[b, pl.ds(eo + j * L, L)]
                    p = p + u * (iv + ev)
                acc = jnp.where(lane == k, jnp.sum(p), acc)
            out_v[pl.ds(c0 + b0, L)] = acc
            return 0

        lax.fori_loop(0, NG, group_body, 0)
        return 0

    lax.fori_loop(0, NCH, chunk_body, 0)
    pltpu.sync_copy(out_v, out_hbm.at[pl.ds(base, BPW)])


def kernel(u_ids, i_ids, ent_map, user_emb, item_emb, ent_emb):
    up = user_emb.reshape(-1, 2 * D)
    ip = item_emb.reshape(-1, 2 * D)
    ep = jnp.pad(ent_emb, ((0, 1), (0, 0))).reshape(-1, 2 * D)

    mesh = plsc.VectorSubcoreMesh(core_axis_name="c", subcore_axis_name="s")
    f = pl.kernel(
        _sc_body,
        out_type=jax.ShapeDtypeStruct((B,), jnp.float32),
        mesh=mesh,
        compiler_params=pltpu.CompilerParams(
            needs_layout_passes=False, use_tc_tiling_on_sc=True),
        scratch_types=[
            pltpu.VMEM((BPW,), jnp.int32),
            pltpu.VMEM((BPW,), jnp.int32),
            pltpu.VMEM((BPW,), jnp.int32),
            pltpu.VMEM((BPW,), jnp.int32),
            pltpu.VMEM((BPW,), jnp.int32),
            pltpu.VMEM((BPW,), jnp.int32),
            pltpu.VMEM((CHUNK, 2 * D), jnp.float32),
            pltpu.VMEM((CHUNK, 2 * D), jnp.float32),
            pltpu.VMEM((CHUNK, 2 * D), jnp.float32),
            pltpu.VMEM((BPW,), jnp.float32),
            pltpu.SemaphoreType.DMA,
            pltpu.SemaphoreType.DMA,
            pltpu.SemaphoreType.DMA,
        ],
    )
    return f(u_ids.astype(jnp.int32), i_ids.astype(jnp.int32),
             ent_map.astype(jnp.int32), up, ip, ep)
